# baseline (speedup 1.0000x reference)
import functools

import jax
import jax.numpy as jnp
from jax import lax
from jax.experimental import pallas as pl
from jax.experimental.pallas import tpu as pltpu
from jax.experimental.pallas import tpu_sc as plsc

VOCAB = 1000000
EMBED_DIM = 32
BATCH = 16384
HIST = 50

_NUM_CORES = 2
_NUM_SUBCORES = 16
_NW = _NUM_CORES * _NUM_SUBCORES

_B = BATCH * HIST
_BPW = BATCH // _NW
_NB = 16
_CIDX = _NB * HIST
_NCHUNK = _BPW // _NB


def _gather_kernel(idx_hbm, table_hbm, out_hbm, idx0, idx1, rows0, rows1,
                   si0, si1, sg0, sg1, ss0, ss1):
    wid = lax.axis_index("s") * _NUM_CORES + lax.axis_index("c")
    base_b = wid * _BPW

    def idx_src(i):
        return idx_hbm.at[pl.ds(base_b + i * _NB, _NB)]

    def out_dst(i):
        return out_hbm.at[pl.ds(base_b + i * _NB, _NB)]

    def start_gathers(idx_v, rows_v, sem):
        for j in range(_NB):
            pltpu.async_copy(
                table_hbm.at[idx_v.at[j]], rows_v.at[j], sem)

    def wait_gathers(idx_v, rows_v, sem):
        for j in range(_NB):
            pltpu.make_async_copy(
                table_hbm.at[idx_v.at[j]], rows_v.at[j], sem).wait()

    pltpu.async_copy(idx_src(0), idx0, si0)
    pltpu.async_copy(idx_src(1), idx1, si1)

    pltpu.make_async_copy(idx_src(0), idx0, si0).wait()
    start_gathers(idx0, rows0, sg0)

    pltpu.make_async_copy(idx_src(1), idx1, si1).wait()
    start_gathers(idx1, rows1, sg1)
    wait_gathers(idx0, rows0, sg0)
    pltpu.async_copy(rows0, out_dst(0), ss0)
    pltpu.async_copy(idx_src(2), idx0, si0)

    def pair_body(g, carry):
        i0 = 2 * g
        i1 = i0 + 1

        pltpu.make_async_copy(idx_src(i0), idx0, si0).wait()
        pltpu.make_async_copy(rows0, out_dst(i0 - 2), ss0).wait()
        start_gathers(idx0, rows0, sg0)

        wait_gathers(idx1, rows1, sg1)
        pltpu.async_copy(rows1, out_dst(i0 - 1), ss1)
        pltpu.async_copy(idx_src(i1), idx1, si1)

        pltpu.make_async_copy(idx_src(i1), idx1, si1).wait()
        pltpu.make_async_copy(rows1, out_dst(i0 - 1), ss1).wait()
        start_gathers(idx1, rows1, sg1)

        wait_gathers(idx0, rows0, sg0)
        pltpu.async_copy(rows0, out_dst(i0), ss0)

        @pl.when(i1 + 1 < _NCHUNK)
        def _():
            pltpu.async_copy(idx_src(i1 + 1), idx0, si0)

        return carry

    lax.fori_loop(1, _NCHUNK // 2, pair_body, 0, unroll=False)

    last = _NCHUNK - 1
    wait_gathers(idx1, rows1, sg1)
    pltpu.async_copy(rows1, out_dst(last), ss1)
    pltpu.make_async_copy(rows0, out_dst(last - 1), ss0).wait()
    pltpu.make_async_copy(rows1, out_dst(last), ss1).wait()


@jax.jit
def _embedding_gather(idx2d, table):
    mesh = plsc.VectorSubcoreMesh(core_axis_name="c", subcore_axis_name="s")
    k = functools.partial(
        pl.kernel,
        mesh=mesh,
        out_type=jax.ShapeDtypeStruct((BATCH, HIST, EMBED_DIM), jnp.float32),
        scratch_types=[
            pltpu.VMEM((_NB, HIST), jnp.int32),
            pltpu.VMEM((_NB, HIST), jnp.int32),
            pltpu.VMEM((_NB, HIST, EMBED_DIM), jnp.float32),
            pltpu.VMEM((_NB, HIST, EMBED_DIM), jnp.float32),
            pltpu.SemaphoreType.DMA,
            pltpu.SemaphoreType.DMA,
            pltpu.SemaphoreType.DMA,
            pltpu.SemaphoreType.DMA,
            pltpu.SemaphoreType.DMA,
            pltpu.SemaphoreType.DMA,
        ],
        compiler_params=pltpu.CompilerParams(use_tc_tiling_on_sc=False),
    )(_gather_kernel)
    return k(idx2d, table)


def kernel(x, table):
    return _embedding_gather(x, table)
